# baseline (speedup 1.0000x reference)
import functools

import jax
import jax.numpy as jnp
from jax import lax
from jax.experimental import pallas as pl
from jax.experimental.pallas import tpu as pltpu
from jax.experimental.pallas import tpu_sc as plsc

DIST_BIN_SIZE = 50
EMB = 16
L = 16
R = 4
RT = 3
RS = R - RT


def kernel(point_dist_mat, extend_point_masks, emb_table):
    B, G, G2 = point_dist_mat.shape
    assert G == G2
    N = B * G * G
    NC, NS = 2, 16
    NW = NC * NS
    rows_total = B * G
    rows_per_w = rows_total // NW
    nblk = rows_per_w // R
    E = R * G
    ES = RS * G

    dist_flat = point_dist_mat.reshape(N)
    mask_i32 = extend_point_masks.astype(jnp.int32)
    table_pad = jnp.zeros((64, EMB), jnp.float32).at[: DIST_BIN_SIZE + 1].set(emb_table)

    mesh = plsc.VectorSubcoreMesh(core_axis_name="c", subcore_axis_name="s")

    @functools.partial(
        pl.kernel,
        out_type=jax.ShapeDtypeStruct((N, EMB), jnp.float32),
        mesh=mesh,
        compiler_params=pltpu.CompilerParams(
            needs_layout_passes=False, use_tc_tiling_on_sc=False),
        scratch_types=[
            pltpu.VMEM_SHARED((64, EMB), jnp.float32),
            pltpu.VMEM((64, EMB), jnp.float32),
            pltpu.VMEM((G,), jnp.int32),
            pltpu.VMEM((E,), jnp.float32),
            pltpu.VMEM((E,), jnp.float32),
            pltpu.VMEM((ES,), jnp.int32),
            pltpu.VMEM((ES,), jnp.int32),
            pltpu.VMEM((E, EMB), jnp.float32),
            pltpu.VMEM((E, EMB), jnp.float32),
            pltpu.SemaphoreType.DMA,
            pltpu.SemaphoreType.DMA,
            pltpu.SemaphoreType.DMA,
            pltpu.SemaphoreType.DMA,
            pltpu.SemaphoreType.DMA,
            pltpu.SemaphoreType.DMA,
        ],
    )
    def run(dist_hbm, mask_hbm, table_hbm, out_hbm,
            table_sh, tabf_v, cmask_v, din0, din1, bins0, bins1, rows0, rows1,
            s_in0, s_in1, s_g0, s_g1, s_out0, s_out1):
        wid = lax.axis_index("s") * NC + lax.axis_index("c")
        row0 = wid * rows_per_w
        b = row0 // G
        i0 = row0 - b * G

        @pl.when(lax.axis_index("s") == 0)
        def _stage_table():
            pltpu.sync_copy(table_hbm, table_sh)

        plsc.subcore_barrier()
        pltpu.sync_copy(table_hbm, tabf_v)
        pltpu.sync_copy(mask_hbm.at[b], cmask_v)

        iota = lax.iota(jnp.int32, L)
        slots = ((din0, bins0, rows0, s_in0, s_g0, s_out0),
                 (din1, bins1, rows1, s_in1, s_g1, s_out1))

        def in_cp(kb, dref, sem):
            src = dist_hbm.at[pl.ds((row0 + kb * R) * G, E)]
            return pltpu.make_async_copy(src, dref, sem)

        def g_cp(binsr, rowsr, sem):
            return pltpu.make_async_copy(
                table_sh.at[binsr], rowsr.at[pl.ds(RT * G, ES)], sem)

        def out_cp(kb, rowsr, sem):
            dst = out_hbm.at[pl.ds((row0 + kb * R) * G, E)]
            return pltpu.make_async_copy(rowsr, dst, sem)

        def make_bins(kb, dinr, r):
            rm = plsc.load_gather(cmask_v, [jnp.full((L,), i0 + kb * R + r, jnp.int32)])
            for v in range(G // L):
                d = dinr[pl.ds(r * G + v * L, L)]
                t = jnp.clip((d / 50.0).astype(jnp.int32), 0, 49)
                tf = t.astype(jnp.float32)
                t = (t - (tf * 50.0 > d).astype(jnp.int32)
                       + ((tf + 1.0) * 50.0 <= d).astype(jnp.int32))
                bv = jnp.minimum(t + 1, DIST_BIN_SIZE - 1)
                cm = cmask_v[pl.ds(v * L, L)]
                yield v, jnp.where((cm | rm) != 0, DIST_BIN_SIZE, bv)

        def compute(kb, dinr, binsr, rowsr):
            def tec_row(r, carry):
                for v, bv in make_bins(kb, dinr, r):
                    perms = [
                        jnp.take_along_axis(
                            bv, jnp.full((L,), lane, jnp.int32), axis=0,
                            mode="promise_in_bounds")
                        for lane in range(L)
                    ]
                    vals = [plsc.load_gather(tabf_v, [p, iota]) for p in perms]
                    for lane in range(L):
                        rowsr[r * G + v * L + lane, :] = vals[lane]
                return carry

            def stream_row(r, carry):
                for v, bv in make_bins(kb, dinr, r):
                    binsr[pl.ds((r - RT) * G + v * L, L)] = bv
                return carry

            lax.fori_loop(0, RT, tec_row, 0)
            lax.fori_loop(RT, R, stream_row, 0)

        in_cp(0, din0, s_in0).start()
        in_cp(1, din1, s_in1).start()

        def body(t, carry):
            for s, (dinr, binsr, rowsr, s_in, s_g, s_out) in enumerate(slots):
                kb = 2 * t + s
                pinr, pbinsr, prowsr, p_in, p_g, p_out = slots[1 - s]
                in_cp(kb, dinr, s_in).wait()

                @pl.when(kb >= 2)
                def _rows_free():
                    out_cp(kb - 2, rowsr, s_out).wait()

                compute(kb, dinr, binsr, rowsr)

                @pl.when(kb + 2 < nblk)
                def _next_in():
                    in_cp(kb + 2, dinr, s_in).start()

                g_cp(binsr, rowsr, s_g).start()

                @pl.when(kb >= 1)
                def _ship_prev():
                    g_cp(pbinsr, prowsr, p_g).wait()
                    out_cp(kb - 1, prowsr, p_out).start()
            return carry

        lax.fori_loop(0, nblk // 2, body, 0)
        g_cp(bins1, rows1, s_g1).wait()
        out_cp(nblk - 1, rows1, s_out1).start()
        out_cp(nblk - 2, rows0, s_out0).wait()
        out_cp(nblk - 1, rows1, s_out1).wait()

    out = run(dist_flat, mask_i32, table_pad)
    return out.reshape(B, G, G * EMB)


# speedup vs baseline: 1.4906x; 1.4906x over previous
import functools

import jax
import jax.numpy as jnp
from jax import lax
from jax.experimental import pallas as pl
from jax.experimental.pallas import tpu as pltpu
from jax.experimental.pallas import tpu_sc as plsc

DIST_BIN_SIZE = 50
EMB = 16
L = 16
R = 4
RT = 2
RS = R - RT


def kernel(point_dist_mat, extend_point_masks, emb_table):
    B, G, G2 = point_dist_mat.shape
    assert G == G2
    N = B * G * G
    NC, NS = 2, 16
    NW = NC * NS
    rows_total = B * G
    rows_per_w = rows_total // NW
    nblk = rows_per_w // R
    E = R * G

    dist_flat = point_dist_mat.reshape(N)
    mask_i32 = extend_point_masks.astype(jnp.int32)
    table_pad = jnp.zeros((64, EMB), jnp.float32).at[: DIST_BIN_SIZE + 1].set(emb_table)

    mesh = plsc.VectorSubcoreMesh(core_axis_name="c", subcore_axis_name="s")

    @functools.partial(
        pl.kernel,
        out_type=jax.ShapeDtypeStruct((N, EMB), jnp.float32),
        mesh=mesh,
        compiler_params=pltpu.CompilerParams(
            needs_layout_passes=False, use_tc_tiling_on_sc=False),
        scratch_types=[
            pltpu.VMEM_SHARED((64, EMB), jnp.float32),
            pltpu.VMEM((64, EMB), jnp.float32),
            pltpu.VMEM((G, EMB), jnp.float32),
            pltpu.VMEM((G,), jnp.int32),
            pltpu.VMEM((E,), jnp.float32),
            pltpu.VMEM((E,), jnp.float32),
            pltpu.VMEM((RS, G), jnp.int32),
            pltpu.VMEM((RS, G), jnp.int32),
            pltpu.VMEM((E, EMB), jnp.float32),
            pltpu.VMEM((E, EMB), jnp.float32),
            pltpu.SemaphoreType.DMA,
            pltpu.SemaphoreType.DMA,
            pltpu.SemaphoreType.DMA,
            pltpu.SemaphoreType.DMA,
            pltpu.SemaphoreType.DMA,
            pltpu.SemaphoreType.DMA,
        ],
    )
    def run(dist_hbm, mask_hbm, table_hbm, out_hbm,
            table_sh, tabf_v, cblk_v, cmask_v, din0, din1, bins0, bins1,
            rows0, rows1, s_in0, s_in1, s_g0, s_g1, s_out0, s_out1):
        wid = lax.axis_index("s") * NC + lax.axis_index("c")
        row0 = wid * rows_per_w
        b = row0 // G
        i0 = row0 - b * G

        @pl.when(lax.axis_index("s") == 0)
        def _stage_table():
            pltpu.sync_copy(table_hbm, table_sh)

        plsc.subcore_barrier()
        pltpu.sync_copy(table_hbm, tabf_v)
        pltpu.sync_copy(mask_hbm.at[b], cmask_v)

        ign = tabf_v[DIST_BIN_SIZE, :]

        def fill_body(j, carry):
            cblk_v[j, :] = ign
            return carry
        lax.fori_loop(0, G, fill_body, 0)

        iota = lax.iota(jnp.int32, L)
        slots = ((din0, bins0, rows0, s_in0, s_g0, s_out0),
                 (din1, bins1, rows1, s_in1, s_g1, s_out1))

        def rmask(kb, j):
            rmv = plsc.load_gather(
                cmask_v, [jnp.full((L,), i0 + kb * R + j, jnp.int32)])
            return jnp.sum(rmv)

        def in_cp(kb, dref, sem):
            src = dist_hbm.at[pl.ds((row0 + kb * R) * G, E)]
            return pltpu.make_async_copy(src, dref, sem)

        def g_row(binsr, rowsr, j, sem):
            return pltpu.make_async_copy(
                table_sh.at[binsr.at[j]],
                rowsr.at[pl.ds((RT + j) * G, G)], sem)

        def out_row(kb, j, src, sem):
            dst = out_hbm.at[pl.ds((row0 + kb * R + j) * G, G)]
            return pltpu.make_async_copy(src, dst, sem)

        def make_bins(kb, dinr, r):
            for v in range(G // L):
                d = dinr[pl.ds(r * G + v * L, L)]
                t = jnp.clip((d / 50.0).astype(jnp.int32), 0, 49)
                tf = t.astype(jnp.float32)
                t = (t - (tf * 50.0 > d).astype(jnp.int32)
                       + ((tf + 1.0) * 50.0 <= d).astype(jnp.int32))
                bv = jnp.minimum(t + 1, DIST_BIN_SIZE - 1)
                cm = cmask_v[pl.ds(v * L, L)]
                yield v, jnp.where(cm != 0, DIST_BIN_SIZE, bv)

        def compute(kb, dinr, binsr, rowsr):
            def tec_row(r, carry):
                @pl.when(rmask(kb, r) == 0)
                def _expand():
                    for v, bv in make_bins(kb, dinr, r):
                        perms = [
                            jnp.take_along_axis(
                                bv, jnp.full((L,), lane, jnp.int32), axis=0,
                                mode="promise_in_bounds")
                            for lane in range(L)
                        ]
                        vals = [plsc.load_gather(tabf_v, [p, iota]) for p in perms]
                        for lane in range(L):
                            rowsr[r * G + v * L + lane, :] = vals[lane]
                return carry

            def stream_row(r, carry):
                @pl.when(rmask(kb, r) == 0)
                def _bins():
                    for v, bv in make_bins(kb, dinr, r):
                        binsr[r - RT, pl.ds(v * L, L)] = bv
                return carry

            lax.fori_loop(0, RT, tec_row, 0)
            lax.fori_loop(RT, R, stream_row, 0)

        def ship_block(kb, binsr, rowsr, s_g, s_out):
            for j in range(RS):
                @pl.when(rmask(kb, RT + j) == 0)
                def _gw(j=j):
                    g_row(binsr, rowsr, j, s_g).wait()
            for j in range(R):
                m = rmask(kb, j)

                @pl.when(m != 0)
                def _const(j=j):
                    out_row(kb, j, cblk_v, s_out).start()

                @pl.when(m == 0)
                def _rows(j=j):
                    out_row(kb, j, rowsr.at[pl.ds(j * G, G)], s_out).start()

        in_cp(0, din0, s_in0).start()
        in_cp(1, din1, s_in1).start()

        def body(t, carry):
            for s, (dinr, binsr, rowsr, s_in, s_g, s_out) in enumerate(slots):
                kb = 2 * t + s
                pinr, pbinsr, prowsr, p_in, p_g, p_out = slots[1 - s]
                in_cp(kb, dinr, s_in).wait()

                @pl.when(kb >= 2)
                def _rows_free():
                    for j in range(R):
                        out_row(kb - 2, j, rowsr.at[pl.ds(j * G, G)], s_out).wait()

                compute(kb, dinr, binsr, rowsr)

                @pl.when(kb + 2 < nblk)
                def _next_in():
                    in_cp(kb + 2, dinr, s_in).start()

                for j in range(RS):
                    @pl.when(rmask(kb, RT + j) == 0)
                    def _gs(j=j):
                        g_row(binsr, rowsr, j, s_g).start()

                @pl.when(kb >= 1)
                def _ship_prev():
                    ship_block(kb - 1, pbinsr, prowsr, p_g, p_out)
            return carry

        lax.fori_loop(0, nblk // 2, body, 0)
        ship_block(nblk - 1, bins1, rows1, s_g1, s_out1)
        for j in range(R):
            out_row(nblk - 2, j, rows0.at[pl.ds(j * G, G)], s_out0).wait()
        for j in range(R):
            out_row(nblk - 1, j, rows1.at[pl.ds(j * G, G)], s_out1).wait()

    out = run(dist_flat, mask_i32, table_pad)
    return out.reshape(B, G, G * EMB)
